# Initial kernel scaffold; baseline (speedup 1.0000x reference)
import jax
import jax.numpy as jnp
from jax.experimental import pallas as pl


def kernel(x, group_centroids, expert_centroids, lb_bias, Wg, Wu, Wd, Sg, Su, Sd):
    raise NotImplementedError("write your pallas kernel here")


# speedup vs baseline: 1.0417x; 1.0417x over previous
import jax
import jax.numpy as jnp
from jax import lax
from jax.experimental import pallas as pl
from jax.experimental.pallas import tpu as pltpu
from jax.experimental.pallas import tpu_sc as plsc

N_TOKENS = 32
D_MODEL = 1024
N_EXPERTS = 16
N_GROUPS = 4
EXPERTS_PER_GROUP = N_EXPERTS // N_GROUPS
TOP_K = 2
N_TOP_GROUPS = 2
D_HID_ROUTED = 2048
D_HID_SHARED = 4096

H_TILE = 512
N_H_TILES = D_HID_ROUTED // H_TILE
N_SHARED_TILES = D_HID_SHARED // H_TILE
N_PSEUDO = N_EXPERTS + N_SHARED_TILES // N_H_TILES

NEG_BIG = -1e30


def _scores_body(x_ref, gcp_ref, ec_ref, lb_ref, gs_ref, es_ref):
    x = x_ref[...]
    xn = x / jnp.maximum(
        jnp.sqrt(jnp.sum(x * x, axis=-1, keepdims=True)), 1e-12)
    gcp = gcp_ref[...]
    gcn = gcp / jnp.maximum(
        jnp.sqrt(jnp.sum(gcp * gcp, axis=-1, keepdims=True)), 1e-12)
    ec = ec_ref[...]
    ecn = ec / jnp.maximum(
        jnp.sqrt(jnp.sum(ec * ec, axis=-1, keepdims=True)), 1e-12)
    gs = lax.dot_general(xn, gcn, (((1,), (1,)), ((), ())),
                         preferred_element_type=jnp.float32)
    col = lax.broadcasted_iota(jnp.int32, (N_TOKENS, N_EXPERTS), 1)
    gs_ref[...] = jnp.where(col < N_GROUPS, gs, NEG_BIG)
    es = lax.dot_general(xn, ecn, (((1,), (1,)), ((), ())),
                         preferred_element_type=jnp.float32)
    es_ref[...] = es + lb_ref[...]


def _routing_scores(x, group_centroids, expert_centroids, lb_bias):
    gc_pad = jnp.zeros((N_EXPERTS, D_MODEL), jnp.float32).at[:N_GROUPS].set(
        group_centroids)
    return pl.pallas_call(
        _scores_body,
        out_shape=(
            jax.ShapeDtypeStruct((N_TOKENS, N_EXPERTS), jnp.float32),
            jax.ShapeDtypeStruct((N_TOKENS, N_EXPERTS), jnp.float32),
        ),
    )(x, gc_pad, expert_centroids, lb_bias.reshape(1, N_EXPERTS))


def _sc_route_body(gs_hbm, es_hbm, out_hbm, gs_v, es_v, rank_v, comb_v):
    wid = lax.axis_index("s") * 2 + lax.axis_index("c")
    pltpu.sync_copy(gs_hbm.at[wid], gs_v)
    pltpu.sync_copy(es_hbm.at[wid], es_v)

    lane = lax.iota(jnp.int32, 16)
    _, gidx = plsc.sort_key_val(gs_v[...], lane, descending=True)
    plsc.store_scatter(rank_v, [gidx], lane)
    grank = plsc.load_gather(rank_v, [lane // EXPERTS_PER_GROUP])
    masked = jnp.where(grank < N_TOP_GROUPS, es_v[...], -1e9)
    sorted_s, eidx = plsc.sort_key_val(masked, lane, descending=True)
    smax = jnp.max(sorted_s)
    e = jnp.where(lane < TOP_K, jnp.exp(sorted_s - smax), 0.0)
    gates = e / jnp.sum(e)
    plsc.store_scatter(comb_v, [eidx], gates)
    pltpu.sync_copy(comb_v, out_hbm.at[wid])


def _sc_route(gs, es):
    mesh = plsc.VectorSubcoreMesh(core_axis_name="c", subcore_axis_name="s")
    return pl.kernel(
        _sc_route_body,
        mesh=mesh,
        compiler_params=pltpu.CompilerParams(needs_layout_passes=False),
        out_type=jax.ShapeDtypeStruct((N_TOKENS, N_EXPERTS), jnp.float32),
        scratch_types=[
            pltpu.VMEM((16,), jnp.float32),
            pltpu.VMEM((16,), jnp.float32),
            pltpu.VMEM((16,), jnp.int32),
            pltpu.VMEM((16,), jnp.float32),
        ],
    )(gs, es)


def _silu(v):
    return v / (1.0 + jnp.exp(-v))


def _experts_body(x_ref, wg_ref, wu_ref, wd_ref, sg_ref, su_ref, sd_ref,
                  p_ref):
    e = pl.program_id(0)
    h = pl.program_id(1)

    @pl.when(h == 0)
    def _():
        p_ref[...] = jnp.zeros_like(p_ref)

    x = x_ref[...]

    @pl.when(e < N_EXPERTS)
    def _():
        hg = jnp.dot(x, wg_ref[0], preferred_element_type=jnp.float32)
        hu = jnp.dot(x, wu_ref[0], preferred_element_type=jnp.float32)
        hsw = _silu(hg) * hu
        p_ref[0] += jnp.dot(hsw, wd_ref[0],
                            preferred_element_type=jnp.float32)

    @pl.when(e >= N_EXPERTS)
    def _():
        hg = jnp.dot(x, sg_ref[...], preferred_element_type=jnp.float32)
        hu = jnp.dot(x, su_ref[...], preferred_element_type=jnp.float32)
        hsw = _silu(hg) * hu
        p_ref[0] += jnp.dot(hsw, sd_ref[...],
                            preferred_element_type=jnp.float32)


def _expert_outputs(x, Wg, Wu, Wd, Sg, Su, Sd):
    def wgu_idx(e, h):
        ec = jnp.minimum(e, N_EXPERTS - 1)
        hc = jnp.where(e < N_EXPERTS, h, N_H_TILES - 1)
        return (ec, 0, hc)

    def wd_idx(e, h):
        ec = jnp.minimum(e, N_EXPERTS - 1)
        hc = jnp.where(e < N_EXPERTS, h, N_H_TILES - 1)
        return (ec, hc, 0)

    def sgu_idx(e, h):
        j = jnp.where(e < N_EXPERTS, 0, (e - N_EXPERTS) * N_H_TILES + h)
        return (0, j)

    def sd_idx(e, h):
        j = jnp.where(e < N_EXPERTS, 0, (e - N_EXPERTS) * N_H_TILES + h)
        return (j, 0)

    return pl.pallas_call(
        _experts_body,
        grid=(N_PSEUDO, N_H_TILES),
        in_specs=[
            pl.BlockSpec((N_TOKENS, D_MODEL), lambda e, h: (0, 0)),
            pl.BlockSpec((1, D_MODEL, H_TILE), wgu_idx),
            pl.BlockSpec((1, D_MODEL, H_TILE), wgu_idx),
            pl.BlockSpec((1, H_TILE, D_MODEL), wd_idx),
            pl.BlockSpec((D_MODEL, H_TILE), sgu_idx),
            pl.BlockSpec((D_MODEL, H_TILE), sgu_idx),
            pl.BlockSpec((H_TILE, D_MODEL), sd_idx),
        ],
        out_specs=pl.BlockSpec((1, N_TOKENS, D_MODEL), lambda e, h: (e, 0, 0)),
        out_shape=jax.ShapeDtypeStruct((N_PSEUDO, N_TOKENS, D_MODEL),
                                       jnp.float32),
    )(x, Wg, Wu, Wd, Sg, Su, Sd)


def _combine_body(p_ref, c_ref, out_ref):
    c = c_ref[...]
    acc = p_ref[N_EXPERTS] + p_ref[N_EXPERTS + 1]
    for e in range(N_EXPERTS):
        acc += c[:, e:e + 1] * p_ref[e]
    out_ref[...] = acc


def _combine(p, comb):
    return pl.pallas_call(
        _combine_body,
        out_shape=jax.ShapeDtypeStruct((N_TOKENS, D_MODEL), jnp.float32),
    )(p, comb)


def kernel(x, group_centroids, expert_centroids, lb_bias, Wg, Wu, Wd, Sg, Su,
           Sd):
    gs, es = _routing_scores(x, group_centroids, expert_centroids, lb_bias)
    comb = _sc_route(gs, es)
    p = _expert_outputs(x, Wg, Wu, Wd, Sg, Su, Sd)
    return _combine(p, comb)


# speedup vs baseline: 1.0761x; 1.0331x over previous
import jax
import jax.numpy as jnp
from jax import lax
from jax.experimental import pallas as pl
from jax.experimental.pallas import tpu as pltpu
from jax.experimental.pallas import tpu_sc as plsc

N_TOKENS = 32
D_MODEL = 1024
N_EXPERTS = 16
N_GROUPS = 4
EXPERTS_PER_GROUP = N_EXPERTS // N_GROUPS
TOP_K = 2
N_TOP_GROUPS = 2
D_HID_ROUTED = 2048
D_HID_SHARED = 4096

H_TILE = 1024
N_H_TILES = D_HID_ROUTED // H_TILE
N_SHARED_TILES = D_HID_SHARED // H_TILE
N_PSEUDO = N_EXPERTS + N_SHARED_TILES // N_H_TILES

NEG_BIG = -1e30


def _scores_body(x_ref, gcp_ref, ec_ref, lb_ref, gs_ref, es_ref):
    x = x_ref[...]
    xn = x / jnp.maximum(
        jnp.sqrt(jnp.sum(x * x, axis=-1, keepdims=True)), 1e-12)
    gcp = gcp_ref[...]
    gcn = gcp / jnp.maximum(
        jnp.sqrt(jnp.sum(gcp * gcp, axis=-1, keepdims=True)), 1e-12)
    ec = ec_ref[...]
    ecn = ec / jnp.maximum(
        jnp.sqrt(jnp.sum(ec * ec, axis=-1, keepdims=True)), 1e-12)
    gs = lax.dot_general(xn, gcn, (((1,), (1,)), ((), ())),
                         preferred_element_type=jnp.float32)
    col = lax.broadcasted_iota(jnp.int32, (N_TOKENS, N_EXPERTS), 1)
    gs_ref[...] = jnp.where(col < N_GROUPS, gs, NEG_BIG)
    es = lax.dot_general(xn, ecn, (((1,), (1,)), ((), ())),
                         preferred_element_type=jnp.float32)
    es_ref[...] = es + lb_ref[...]


def _routing_scores(x, group_centroids, expert_centroids, lb_bias):
    gc_pad = jnp.zeros((N_EXPERTS, D_MODEL), jnp.float32).at[:N_GROUPS].set(
        group_centroids)
    return pl.pallas_call(
        _scores_body,
        out_shape=(
            jax.ShapeDtypeStruct((N_TOKENS, N_EXPERTS), jnp.float32),
            jax.ShapeDtypeStruct((N_TOKENS, N_EXPERTS), jnp.float32),
        ),
    )(x, gc_pad, expert_centroids, lb_bias.reshape(1, N_EXPERTS))


def _sc_route_body(gs_hbm, es_hbm, out_hbm, gs_v, es_v, rank_v, comb_v):
    wid = lax.axis_index("s") * 2 + lax.axis_index("c")
    pltpu.sync_copy(gs_hbm.at[wid], gs_v)
    pltpu.sync_copy(es_hbm.at[wid], es_v)

    lane = lax.iota(jnp.int32, 16)
    _, gidx = plsc.sort_key_val(gs_v[...], lane, descending=True)
    plsc.store_scatter(rank_v, [gidx], lane)
    grank = plsc.load_gather(rank_v, [lane // EXPERTS_PER_GROUP])
    masked = jnp.where(grank < N_TOP_GROUPS, es_v[...], -1e9)
    sorted_s, eidx = plsc.sort_key_val(masked, lane, descending=True)
    smax = jnp.max(sorted_s)
    e = jnp.where(lane < TOP_K, jnp.exp(sorted_s - smax), 0.0)
    gates = e / jnp.sum(e)
    plsc.store_scatter(comb_v, [eidx], gates)
    pltpu.sync_copy(comb_v, out_hbm.at[wid])


def _sc_route(gs, es):
    mesh = plsc.VectorSubcoreMesh(core_axis_name="c", subcore_axis_name="s")
    return pl.kernel(
        _sc_route_body,
        mesh=mesh,
        compiler_params=pltpu.CompilerParams(needs_layout_passes=False),
        out_type=jax.ShapeDtypeStruct((N_TOKENS, N_EXPERTS), jnp.float32),
        scratch_types=[
            pltpu.VMEM((16,), jnp.float32),
            pltpu.VMEM((16,), jnp.float32),
            pltpu.VMEM((16,), jnp.int32),
            pltpu.VMEM((16,), jnp.float32),
        ],
    )(gs, es)


def _silu(v):
    return v / (1.0 + jnp.exp(-v))


def _experts_body(x_ref, wg_ref, wu_ref, wd_ref, sg_ref, su_ref, sd_ref,
                  p_ref):
    e = pl.program_id(0)
    h = pl.program_id(1)

    @pl.when(h == 0)
    def _():
        p_ref[...] = jnp.zeros_like(p_ref)

    x = x_ref[...]

    @pl.when(e < N_EXPERTS)
    def _():
        hg = jnp.dot(x, wg_ref[0], preferred_element_type=jnp.float32)
        hu = jnp.dot(x, wu_ref[0], preferred_element_type=jnp.float32)
        hsw = _silu(hg) * hu
        p_ref[0] += jnp.dot(hsw, wd_ref[0],
                            preferred_element_type=jnp.float32)

    @pl.when(e >= N_EXPERTS)
    def _():
        hg = jnp.dot(x, sg_ref[...], preferred_element_type=jnp.float32)
        hu = jnp.dot(x, su_ref[...], preferred_element_type=jnp.float32)
        hsw = _silu(hg) * hu
        p_ref[0] += jnp.dot(hsw, sd_ref[...],
                            preferred_element_type=jnp.float32)


def _expert_outputs(x, Wg, Wu, Wd, Sg, Su, Sd):
    def wgu_idx(e, h):
        ec = jnp.minimum(e, N_EXPERTS - 1)
        hc = jnp.where(e < N_EXPERTS, h, N_H_TILES - 1)
        return (ec, 0, hc)

    def wd_idx(e, h):
        ec = jnp.minimum(e, N_EXPERTS - 1)
        hc = jnp.where(e < N_EXPERTS, h, N_H_TILES - 1)
        return (ec, hc, 0)

    def sgu_idx(e, h):
        j = jnp.where(e < N_EXPERTS, 0, (e - N_EXPERTS) * N_H_TILES + h)
        return (0, j)

    def sd_idx(e, h):
        j = jnp.where(e < N_EXPERTS, 0, (e - N_EXPERTS) * N_H_TILES + h)
        return (j, 0)

    return pl.pallas_call(
        _experts_body,
        grid=(N_PSEUDO, N_H_TILES),
        in_specs=[
            pl.BlockSpec((N_TOKENS, D_MODEL), lambda e, h: (0, 0)),
            pl.BlockSpec((1, D_MODEL, H_TILE), wgu_idx),
            pl.BlockSpec((1, D_MODEL, H_TILE), wgu_idx),
            pl.BlockSpec((1, H_TILE, D_MODEL), wd_idx),
            pl.BlockSpec((D_MODEL, H_TILE), sgu_idx),
            pl.BlockSpec((D_MODEL, H_TILE), sgu_idx),
            pl.BlockSpec((H_TILE, D_MODEL), sd_idx),
        ],
        out_specs=pl.BlockSpec((1, N_TOKENS, D_MODEL), lambda e, h: (e, 0, 0)),
        out_shape=jax.ShapeDtypeStruct((N_PSEUDO, N_TOKENS, D_MODEL),
                                       jnp.float32),
    )(x, Wg, Wu, Wd, Sg, Su, Sd)


def _combine_body(p_ref, c_ref, out_ref):
    c = c_ref[...]
    acc = p_ref[N_EXPERTS] + p_ref[N_EXPERTS + 1]
    for e in range(N_EXPERTS):
        acc += c[:, e:e + 1] * p_ref[e]
    out_ref[...] = acc


def _combine(p, comb):
    return pl.pallas_call(
        _combine_body,
        out_shape=jax.ShapeDtypeStruct((N_TOKENS, D_MODEL), jnp.float32),
    )(p, comb)


def kernel(x, group_centroids, expert_centroids, lb_bias, Wg, Wu, Wd, Sg, Su,
           Sd):
    gs, es = _routing_scores(x, group_centroids, expert_centroids, lb_bias)
    comb = _sc_route(gs, es)
    p = _expert_outputs(x, Wg, Wu, Wd, Sg, Su, Sd)
    return _combine(p, comb)
